# baseline (speedup 1.0000x reference)
import jax
import jax.numpy as jnp
from jax.experimental import pallas as pl
from jax.experimental.pallas import tpu as pltpu

_N_BINS = 15
_ROWS, _COLS = 16384, 2048
_BLK = 256


def _fused_kernel(temp_ref, bc_ref, bt_ref, x_ref,
                  xout_ref, ece_ref, tout_ref, acc_ref):
    xout_ref[...] = x_ref[...]

    @pl.when(pl.program_id(0) == 0)
    def _stats():
        n = jnp.float32(0.0)
        for i in range(_N_BINS):
            n = n + bt_ref[i]
        s = jnp.float32(0.0)
        for i in range(_N_BINS):
            bc = bc_ref[i]
            bt = bt_ref[i]
            acc = bc / (bt + 1e-8)
            acc_ref[i] = acc
            conf = i / (_N_BINS - 1.0) + 0.5 / _N_BINS
            s = s + bt * jnp.abs(acc - conf)
        ece_ref[0] = jnp.where(n > 0.0, s / jnp.maximum(n, 1e-8), 0.0)
        tout_ref[0] = jnp.clip(temp_ref[0], 0.1, 10.0)


def kernel(x, temperature, platt_a, platt_b, bin_correct, bin_total):
    xout, ece, temp, acc = pl.pallas_call(
        _fused_kernel,
        grid=(_ROWS // _BLK,),
        out_shape=(
            jax.ShapeDtypeStruct((_ROWS, _COLS), jnp.float32),
            jax.ShapeDtypeStruct((1,), jnp.float32),
            jax.ShapeDtypeStruct((1,), jnp.float32),
            jax.ShapeDtypeStruct((_N_BINS,), jnp.float32),
        ),
        in_specs=[
            pl.BlockSpec(memory_space=pltpu.SMEM),
            pl.BlockSpec(memory_space=pltpu.SMEM),
            pl.BlockSpec(memory_space=pltpu.SMEM),
            pl.BlockSpec((_BLK, _COLS), lambda i: (i, 0)),
        ],
        out_specs=(
            pl.BlockSpec((_BLK, _COLS), lambda i: (i, 0)),
            pl.BlockSpec(memory_space=pltpu.SMEM),
            pl.BlockSpec(memory_space=pltpu.SMEM),
            pl.BlockSpec(memory_space=pltpu.SMEM),
        ),
    )(temperature.reshape(1), bin_correct, bin_total, x)
    return (xout, ece.reshape(()), temp.reshape(()), acc)


# speedup vs baseline: 1.1145x; 1.1145x over previous
import jax
import jax.numpy as jnp
from jax.experimental import pallas as pl
from jax.experimental.pallas import tpu as pltpu

_N_BINS = 15
_ROWS, _COLS = 16384, 2048
_BLK = 1024


def _fused_kernel(temp_ref, bc_ref, bt_ref, x_ref,
                  xout_ref, ece_ref, tout_ref, acc_ref):
    xout_ref[...] = x_ref[...]

    @pl.when(pl.program_id(0) == 0)
    def _stats():
        n = jnp.float32(0.0)
        for i in range(_N_BINS):
            n = n + bt_ref[i]
        s = jnp.float32(0.0)
        for i in range(_N_BINS):
            bc = bc_ref[i]
            bt = bt_ref[i]
            acc = bc / (bt + 1e-8)
            acc_ref[i] = acc
            conf = i / (_N_BINS - 1.0) + 0.5 / _N_BINS
            s = s + bt * jnp.abs(acc - conf)
        ece_ref[0] = jnp.where(n > 0.0, s / jnp.maximum(n, 1e-8), 0.0)
        tout_ref[0] = jnp.clip(temp_ref[0], 0.1, 10.0)


def kernel(x, temperature, platt_a, platt_b, bin_correct, bin_total):
    xout, ece, temp, acc = pl.pallas_call(
        _fused_kernel,
        grid=(_ROWS // _BLK,),
        out_shape=(
            jax.ShapeDtypeStruct((_ROWS, _COLS), jnp.float32),
            jax.ShapeDtypeStruct((1,), jnp.float32),
            jax.ShapeDtypeStruct((1,), jnp.float32),
            jax.ShapeDtypeStruct((_N_BINS,), jnp.float32),
        ),
        in_specs=[
            pl.BlockSpec(memory_space=pltpu.SMEM),
            pl.BlockSpec(memory_space=pltpu.SMEM),
            pl.BlockSpec(memory_space=pltpu.SMEM),
            pl.BlockSpec((_BLK, _COLS), lambda i: (i, 0)),
        ],
        out_specs=(
            pl.BlockSpec((_BLK, _COLS), lambda i: (i, 0)),
            pl.BlockSpec(memory_space=pltpu.SMEM),
            pl.BlockSpec(memory_space=pltpu.SMEM),
            pl.BlockSpec(memory_space=pltpu.SMEM),
        ),
    )(temperature.reshape(1), bin_correct, bin_total, x)
    return (xout, ece.reshape(()), temp.reshape(()), acc)
